# baseline (speedup 1.0000x reference)
import functools

import jax
import jax.numpy as jnp
from jax import lax
from jax.experimental import pallas as pl
from jax.experimental.pallas import tpu as pltpu
from jax.experimental.pallas import tpu_sc as plsc

B, L, V, D = 16384, 200, 8, 2
N = B * L
NUM_WORKERS = 32
LT = L // 8
RT = B // 128
TQ = RT // NUM_WORKERS
NBLK = LT
IN_BLK = TQ * 1024
OUT_ROW = TQ * D * 128
CHUNKS = IN_BLK // 16


def _sc_body(emb_hbm, w_hbm, b_hbm, x_hbm, out_hbm,
             xin0, xin1, ov0, ov1, tbl, pe, pw, pb, sems):
    xin = [xin0, xin1]
    ov = [ov0, ov1]
    wid = lax.axis_index("s") * 2 + lax.axis_index("c")

    def _issue_in_first(blk, buf):
        off = (blk * RT + TQ * wid) * 1024
        pltpu.async_copy(x_hbm.at[pl.ds(off, IN_BLK)], xin[buf], sems.at[buf])

    _issue_in_first(0, 0)
    _issue_in_first(1, 1)

    pltpu.sync_copy(emb_hbm, pe)
    pltpu.sync_copy(w_hbm, pw)
    pltpu.sync_copy(b_hbm, pb)
    iota = lax.iota(jnp.int32, 16)
    v = iota >> 1
    c = iota & 1
    zero = iota * 0
    one = zero + 1
    e0 = plsc.load_gather(pe, [v, zero])
    e1 = plsc.load_gather(pe, [v, one])
    w0 = plsc.load_gather(pw, [c, zero])
    w1 = plsc.load_gather(pw, [c, one])
    bb = plsc.load_gather(pb, [c])
    tbl[...] = e0 * w0 + e1 * w1 + bb

    def issue_in(blk, buf):
        off = (blk * RT + TQ * wid) * 1024
        pltpu.async_copy(x_hbm.at[pl.ds(off, IN_BLK)], xin[buf], sems.at[buf])

    def wait_in(buf):
        pltpu.make_async_copy(
            x_hbm.at[pl.ds(0, IN_BLK)], xin[buf], sems.at[buf]).wait()

    def issue_out(blk, buf):
        for lm in range(8):
            off = ((8 * blk + lm) * RT + TQ * wid) * (D * 128)
            pltpu.async_copy(ov[buf].at[pl.ds(OUT_ROW * lm, OUT_ROW)],
                             out_hbm.at[pl.ds(off, OUT_ROW)],
                             sems.at[2 + buf])

    def wait_out(buf):
        for lm in range(8):
            pltpu.make_async_copy(
                ov[buf].at[pl.ds(OUT_ROW * lm, OUT_ROW)],
                out_hbm.at[pl.ds(OUT_ROW * lm, OUT_ROW)],
                sems.at[2 + buf]).wait()

    def compute(buf):
        xbuf = xin[buf]
        obuf = ov[buf]

        @plsc.parallel_loop(0, CHUNKS, 1, unroll=8)
        def chunk(ci):
            xv = xbuf[pl.ds(ci * 16, 16)]
            i0 = xv * 2
            g0 = plsc.load_gather(tbl, [i0])
            g1 = plsc.load_gather(tbl, [i0 + 1])
            t = ci >> 6
            kk = ci & 63
            pos = (kk >> 3) * 1024 + t * 256 + (kk & 7) * 16
            obuf[pl.ds(pos, 16)] = g0
            obuf[pl.ds(pos + 128, 16)] = g1

    def body(k, carry):
        b0 = 2 * k

        wait_in(0)

        @pl.when(k > 0)
        def _():
            wait_out(0)

        compute(0)
        issue_out(b0, 0)
        issue_in(b0 + 2, 0)

        wait_in(1)

        @pl.when(k > 0)
        def _():
            wait_out(1)

        compute(1)
        issue_out(b0 + 1, 1)

        @pl.when(k < (NBLK - 1) // 2 - 1)
        def _():
            issue_in(b0 + 3, 1)

        return carry

    lax.fori_loop(0, (NBLK - 1) // 2, body, 0)

    wait_in(0)
    wait_out(0)
    compute(0)
    issue_out(NBLK - 1, 0)
    wait_out(1)
    wait_out(0)


@jax.jit
def _sc_lookup(emb, W, b, xq):
    mesh = plsc.VectorSubcoreMesh(core_axis_name="c", subcore_axis_name="s")
    f = pl.kernel(
        _sc_body,
        out_type=jax.ShapeDtypeStruct((D * N,), jnp.float32),
        mesh=mesh,
        compiler_params=pltpu.CompilerParams(
            needs_layout_passes=False,
            disable_bounds_checks=True,
            disable_semaphore_checks=True,
            skip_device_barrier=True,
        ),
        scratch_types=[
            pltpu.VMEM((IN_BLK,), jnp.int32),
            pltpu.VMEM((IN_BLK,), jnp.int32),
            pltpu.VMEM((8 * OUT_ROW,), jnp.float32),
            pltpu.VMEM((8 * OUT_ROW,), jnp.float32),
            pltpu.VMEM((16,), jnp.float32),
            pltpu.VMEM((V, D), jnp.float32),
            pltpu.VMEM((D, D), jnp.float32),
            pltpu.VMEM((D,), jnp.float32),
            pltpu.SemaphoreType.DMA((4,)),
        ],
    )
    return f(emb, W, b, xq)


def kernel(x, emb, W, b):
    xq = (x.astype(jnp.int32).T
          .reshape(LT, 8, RT, 128).transpose(0, 2, 1, 3).reshape(-1))
    of = _sc_lookup(emb, W, b, xq)
    return of.reshape(L, RT, D, 128).transpose(1, 3, 0, 2).reshape(B, L, D)


# speedup vs baseline: 1.0576x; 1.0576x over previous
import functools

import jax
import jax.numpy as jnp
from jax import lax
from jax.experimental import pallas as pl
from jax.experimental.pallas import tpu as pltpu
from jax.experimental.pallas import tpu_sc as plsc

B, L, V, D = 16384, 200, 8, 2
N = B * L
NUM_WORKERS = 32
LT = L // 8
RT = B // 128
TQ = RT // NUM_WORKERS
NBLK = LT
IN_BLK = TQ * 1024
OUT_ROW = TQ * D * 128
CHUNKS = IN_BLK // 16


def _sc_body(params_hbm, x_hbm, out_hbm,
             xin0, xin1, ov0, ov1, tbl, pe, pw, sems):
    xin = [xin0, xin1]
    ov = [ov0, ov1]
    wid = lax.axis_index("s") * 2 + lax.axis_index("c")

    def _issue_in_first(blk, buf):
        off = (blk * RT + TQ * wid) * 1024
        pltpu.async_copy(x_hbm.at[pl.ds(off, IN_BLK)], xin[buf], sems.at[buf])

    _issue_in_first(0, 0)
    _issue_in_first(1, 1)

    pltpu.sync_copy(params_hbm.at[pl.ds(0, 16)], pe)
    pltpu.sync_copy(params_hbm.at[pl.ds(16, 16)], pw)
    iota = lax.iota(jnp.int32, 16)
    v = iota >> 1
    c = iota & 1
    e0 = plsc.load_gather(pe, [v * 2])
    e1 = plsc.load_gather(pe, [v * 2 + 1])
    w0 = plsc.load_gather(pw, [c * 2])
    w1 = plsc.load_gather(pw, [c * 2 + 1])
    bb = plsc.load_gather(pw, [c + 4])
    tbl[...] = e0 * w0 + e1 * w1 + bb

    def issue_in(blk, buf):
        off = (blk * RT + TQ * wid) * 1024
        pltpu.async_copy(x_hbm.at[pl.ds(off, IN_BLK)], xin[buf], sems.at[buf])

    def wait_in(buf):
        pltpu.make_async_copy(
            x_hbm.at[pl.ds(0, IN_BLK)], xin[buf], sems.at[buf]).wait()

    def issue_out(blk, buf):
        for lm in range(8):
            off = ((8 * blk + lm) * RT + TQ * wid) * (D * 128)
            pltpu.async_copy(ov[buf].at[pl.ds(OUT_ROW * lm, OUT_ROW)],
                             out_hbm.at[pl.ds(off, OUT_ROW)],
                             sems.at[2 + buf])

    def wait_out(buf):
        for lm in range(8):
            pltpu.make_async_copy(
                ov[buf].at[pl.ds(OUT_ROW * lm, OUT_ROW)],
                out_hbm.at[pl.ds(OUT_ROW * lm, OUT_ROW)],
                sems.at[2 + buf]).wait()

    def compute(buf):
        xbuf = xin[buf]
        obuf = ov[buf]

        @plsc.parallel_loop(0, CHUNKS, 1, unroll=8)
        def chunk(ci):
            xv = xbuf[pl.ds(ci * 16, 16)]
            i0 = xv * 2
            g0 = plsc.load_gather(tbl, [i0])
            g1 = plsc.load_gather(tbl, [i0 + 1])
            t = ci >> 6
            kk = ci & 63
            pos = (kk >> 3) * 1024 + t * 256 + (kk & 7) * 16
            obuf[pl.ds(pos, 16)] = g0
            obuf[pl.ds(pos + 128, 16)] = g1

    def body(k, carry):
        b0 = 2 * k

        wait_in(0)

        @pl.when(k > 0)
        def _():
            wait_out(0)

        compute(0)
        issue_out(b0, 0)
        issue_in(b0 + 2, 0)

        wait_in(1)

        @pl.when(k > 0)
        def _():
            wait_out(1)

        compute(1)
        issue_out(b0 + 1, 1)

        @pl.when(k < (NBLK - 1) // 2 - 1)
        def _():
            issue_in(b0 + 3, 1)

        return carry

    lax.fori_loop(0, (NBLK - 1) // 2, body, 0)

    wait_in(0)
    wait_out(0)
    compute(0)
    issue_out(NBLK - 1, 0)
    wait_out(1)
    wait_out(0)


@jax.jit
def _sc_lookup(params, xq):
    mesh = plsc.VectorSubcoreMesh(core_axis_name="c", subcore_axis_name="s")
    f = pl.kernel(
        _sc_body,
        out_type=jax.ShapeDtypeStruct((D * N,), jnp.float32),
        mesh=mesh,
        compiler_params=pltpu.CompilerParams(
            needs_layout_passes=False,
            disable_bounds_checks=True,
            disable_semaphore_checks=True,
            skip_device_barrier=True,
        ),
        scratch_types=[
            pltpu.VMEM((IN_BLK,), jnp.int32),
            pltpu.VMEM((IN_BLK,), jnp.int32),
            pltpu.VMEM((8 * OUT_ROW,), jnp.float32),
            pltpu.VMEM((8 * OUT_ROW,), jnp.float32),
            pltpu.VMEM((16,), jnp.float32),
            pltpu.VMEM((16,), jnp.float32),
            pltpu.VMEM((16,), jnp.float32),
            pltpu.SemaphoreType.DMA((4,)),
        ],
    )
    return f(params, xq)


def kernel(x, emb, W, b):
    params = jnp.concatenate(
        [emb.reshape(-1), W.reshape(-1), b.reshape(-1),
         jnp.zeros((10,), jnp.float32)])
    xq = (x.astype(jnp.int32).T
          .reshape(LT, 8, RT, 128).transpose(0, 2, 1, 3).reshape(-1))
    of = _sc_lookup(params, xq)
    return of.reshape(L, RT, D, 128).transpose(1, 3, 0, 2).reshape(B, L, D)
